# Initial kernel scaffold; baseline (speedup 1.0000x reference)
import jax
import jax.numpy as jnp
from jax.experimental import pallas as pl


def kernel(x, edge_index, W_l1, W_r1, b1, a1, W_l2, W_r2, b2, a2):
    raise NotImplementedError("write your pallas kernel here")


# speedup vs baseline: 5.1480x; 5.1480x over previous
import functools

import jax
import jax.numpy as jnp
from jax import lax
from jax.experimental import pallas as pl
from jax.experimental.pallas import tpu as pltpu
from jax.experimental.pallas import tpu_sc as plsc

N = 10000
E = 320000
D = 128

NC = 2
NS = 16
NW = NC * NS
BLK = 128

EPW = ((E + NW - 1) // NW + BLK - 1) // BLK * BLK
NBLK = EPW // BLK
E_PAD = EPW * NW

ROWS_PER_TILE = 640
N_PAD = NS * ROWS_PER_TILE
ZBLK = ROWS_PER_TILE // BLK


def _seg_body(x_hbm, src_hbm, dst_hbm, acc_out, cnt_out,
              acc_sh, cnt_sh, srcb, dstb, rows, ones, zcnt):
    cid = lax.axis_index("c")
    sid = lax.axis_index("s")
    wid = sid * NC + cid
    zero16 = jnp.zeros((16,), jnp.float32)
    one16 = jnp.ones((16,), jnp.float32)

    for i in range(BLK // 16):
        ones[pl.ds(i * 16, 16)] = one16
    for i in range(ROWS_PER_TILE // 16):
        zcnt[pl.ds(i * 16, 16)] = zero16

    def _zrow(r, carry):
        for c in range(D // 16):
            rows[r, pl.ds(c * 16, 16)] = zero16
        return carry
    lax.fori_loop(0, BLK, _zrow, 0)
    base_row = sid * ROWS_PER_TILE
    for j in range(ZBLK):
        pltpu.sync_copy(rows, acc_sh.at[pl.ds(base_row + j * BLK, BLK)])
    pltpu.sync_copy(zcnt, cnt_sh.at[pl.ds(base_row, ROWS_PER_TILE)])
    plsc.subcore_barrier()

    pltpu.sync_copy(src_hbm.at[wid], srcb)
    pltpu.sync_copy(dst_hbm.at[wid], dstb)

    def _step(b, carry):
        sidx = srcb.at[b]
        didx = dstb.at[b]
        pltpu.sync_copy(x_hbm.at[sidx], rows)
        pltpu.sync_copy(rows, acc_sh.at[didx], add=True)
        pltpu.sync_copy(ones, cnt_sh.at[didx], add=True)
        return carry
    lax.fori_loop(0, NBLK, _step, 0)
    plsc.subcore_barrier()

    for j in range(ZBLK):
        r = base_row + j * BLK
        pltpu.sync_copy(acc_sh.at[pl.ds(r, BLK)], acc_out.at[cid, pl.ds(r, BLK)])
    pltpu.sync_copy(cnt_sh.at[pl.ds(base_row, ROWS_PER_TILE)],
                    cnt_out.at[cid, pl.ds(base_row, ROWS_PER_TILE)])


_seg_sum = pl.kernel(
    _seg_body,
    out_type=[
        jax.ShapeDtypeStruct((NC, N_PAD, D), jnp.float32),
        jax.ShapeDtypeStruct((NC, N_PAD), jnp.float32),
    ],
    mesh=plsc.VectorSubcoreMesh(core_axis_name="c", subcore_axis_name="s",
                                num_cores=NC, num_subcores=NS),
    scratch_types=[
        pltpu.VMEM_SHARED((N_PAD, D), jnp.float32),
        pltpu.VMEM_SHARED((N_PAD,), jnp.float32),
        pltpu.VMEM((NBLK, BLK), jnp.int32),
        pltpu.VMEM((NBLK, BLK), jnp.int32),
        pltpu.VMEM((BLK, D), jnp.float32),
        pltpu.VMEM((BLK,), jnp.float32),
        pltpu.VMEM((ROWS_PER_TILE,), jnp.float32),
    ],
)


def _dense_body(acc_ref, cnt_ref, x_ref, wl_ref, wr_ref, b_ref, a_ref, o_ref):
    cnt = cnt_ref[0] + cnt_ref[1]
    mean = (acc_ref[0] + acc_ref[1]) / jnp.maximum(cnt, 1.0)
    h = lax.dot_general(mean, wl_ref[...], (((1,), (1,)), ((), ())),
                        preferred_element_type=jnp.float32)
    h = h + lax.dot_general(x_ref[...], wr_ref[...], (((1,), (1,)), ((), ())),
                            preferred_element_type=jnp.float32)
    h = h + b_ref[...]
    a = a_ref[0, 0]
    o_ref[...] = jnp.where(h > 0.0, h, a * h)


_RB = 400

_dense = pl.pallas_call(
    _dense_body,
    grid=(N // _RB,),
    in_specs=[
        pl.BlockSpec((NC, _RB, D), lambda i: (0, i, 0)),
        pl.BlockSpec((NC, _RB, 1), lambda i: (0, i, 0)),
        pl.BlockSpec((_RB, D), lambda i: (i, 0)),
        pl.BlockSpec((D, D), lambda i: (0, 0)),
        pl.BlockSpec((D, D), lambda i: (0, 0)),
        pl.BlockSpec((1, D), lambda i: (0, 0)),
        pl.BlockSpec((1, 1), lambda i: (0, 0)),
    ],
    out_specs=pl.BlockSpec((_RB, D), lambda i: (i, 0)),
    out_shape=jax.ShapeDtypeStruct((N, D), jnp.float32),
)


def kernel(x, edge_index, W_l1, W_r1, b1, a1, W_l2, W_r2, b2, a2):
    h0 = x[0]
    ei = edge_index[0].astype(jnp.int32)
    pad = E_PAD - E
    src = jnp.concatenate([ei[0], jnp.zeros((pad,), jnp.int32)]).reshape(NW, NBLK, BLK)
    dst = jnp.concatenate([ei[1], jnp.full((pad,), N, jnp.int32)]).reshape(NW, NBLK, BLK)

    acc1, cnt1 = _seg_sum(h0, src, dst)
    h1 = _dense(acc1, cnt1.reshape(NC, N_PAD, 1), h0, W_l1, W_r1,
                b1.reshape(1, D), a1.reshape(1, 1))
    acc2, cnt2 = _seg_sum(h1, src, dst)
    h2 = _dense(acc2, cnt2.reshape(NC, N_PAD, 1), h1, W_l2, W_r2,
                b2.reshape(1, D), a2.reshape(1, 1))
    return h2.reshape(x.shape[:-2] + (-1,))


# speedup vs baseline: 7.6626x; 1.4885x over previous
import functools

import jax
import jax.numpy as jnp
from jax import lax
from jax.experimental import pallas as pl
from jax.experimental.pallas import tpu as pltpu
from jax.experimental.pallas import tpu_sc as plsc

N = 10000
E = 320000
D = 128

NC = 2
NS = 16
NW = NC * NS
BLK = 64

EPW = ((E + NW - 1) // NW + BLK - 1) // BLK * BLK
NBLK = EPW // BLK
E_PAD = EPW * NW

ROWS_PER_TILE = 640
N_PAD = NS * ROWS_PER_TILE
ZBLK = ROWS_PER_TILE // BLK


def _seg_body(x_hbm, ei_hbm, acc_out, cnt_out,
              acc_sh, cnt_sh, idxb, rows, ones, zcnt, isem, gsem):
    cid = lax.axis_index("c")
    sid = lax.axis_index("s")
    wid = sid * NC + cid
    zero16 = jnp.zeros((16,), jnp.float32)
    one16 = jnp.ones((16,), jnp.float32)

    for i in range(BLK // 16):
        ones[pl.ds(i * 16, 16)] = one16
    for i in range(ROWS_PER_TILE // 16):
        zcnt[pl.ds(i * 16, 16)] = zero16

    def _zrow(r, carry):
        for c in range(D // 16):
            rows[0, r, pl.ds(c * 16, 16)] = zero16
        return carry
    lax.fori_loop(0, BLK, _zrow, 0)
    base_row = sid * ROWS_PER_TILE
    for j in range(ZBLK):
        pltpu.sync_copy(rows.at[0], acc_sh.at[pl.ds(base_row + j * BLK, BLK)])
    pltpu.sync_copy(zcnt, cnt_sh.at[pl.ds(base_row, ROWS_PER_TILE)])
    plsc.subcore_barrier()

    def _prefetch(b, q):
        pltpu.async_copy(ei_hbm.at[wid, b], idxb.at[q], isem.at[q])

    def _wait_idx(q):
        pltpu.make_async_copy(ei_hbm.at[wid, 0], idxb.at[q], isem.at[q]).wait()

    def _gather(q, p):
        pltpu.async_copy(x_hbm.at[idxb.at[q, 0]], rows.at[p], gsem.at[p])

    def _wait_rows(p):
        pltpu.make_async_copy(x_hbm.at[pl.ds(0, BLK)], rows.at[p], gsem.at[p]).wait()

    for q in range(4):
        _prefetch(q, q)
    _wait_idx(0)
    _gather(0, 0)
    _wait_idx(1)
    _gather(1, 1)

    def _step(b, carry):
        p = lax.rem(b, 2)
        q = lax.rem(b, 4)
        didx = idxb.at[q, 1]
        _wait_rows(p)
        pltpu.sync_copy(rows.at[p], acc_sh.at[didx], add=True)
        pltpu.sync_copy(ones, cnt_sh.at[didx], add=True)

        @pl.when(b < NBLK - 2)
        def _():
            q2 = lax.rem(b + 2, 4)
            _wait_idx(q2)
            _gather(q2, p)

        @pl.when(b < NBLK - 4)
        def _():
            _prefetch(b + 4, lax.rem(b + 4, 4))
        return carry
    lax.fori_loop(0, NBLK, _step, 0)
    plsc.subcore_barrier()

    for j in range(ZBLK):
        r = base_row + j * BLK
        pltpu.sync_copy(acc_sh.at[pl.ds(r, BLK)], acc_out.at[cid, pl.ds(r, BLK)])
    pltpu.sync_copy(cnt_sh.at[pl.ds(base_row, ROWS_PER_TILE)],
                    cnt_out.at[cid, pl.ds(base_row, ROWS_PER_TILE)])


_seg_sum = pl.kernel(
    _seg_body,
    out_type=[
        jax.ShapeDtypeStruct((NC, N_PAD, D), jnp.float32),
        jax.ShapeDtypeStruct((NC, N_PAD), jnp.float32),
    ],
    mesh=plsc.VectorSubcoreMesh(core_axis_name="c", subcore_axis_name="s",
                                num_cores=NC, num_subcores=NS),
    scratch_types=[
        pltpu.VMEM_SHARED((N_PAD, D), jnp.float32),
        pltpu.VMEM_SHARED((N_PAD,), jnp.float32),
        pltpu.VMEM((4, 2, BLK), jnp.int32),
        pltpu.VMEM((2, BLK, D), jnp.float32),
        pltpu.VMEM((BLK,), jnp.float32),
        pltpu.VMEM((ROWS_PER_TILE,), jnp.float32),
        pltpu.SemaphoreType.DMA((4,)),
        pltpu.SemaphoreType.DMA((2,)),
    ],
)


def _dense_body(acc_ref, cnt_ref, x_ref, wl_ref, wr_ref, b_ref, a_ref, o_ref):
    cnt = cnt_ref[0] + cnt_ref[1]
    mean = (acc_ref[0] + acc_ref[1]) / jnp.maximum(cnt, 1.0)
    h = lax.dot_general(mean, wl_ref[...], (((1,), (1,)), ((), ())),
                        preferred_element_type=jnp.float32)
    h = h + lax.dot_general(x_ref[...], wr_ref[...], (((1,), (1,)), ((), ())),
                            preferred_element_type=jnp.float32)
    h = h + b_ref[...]
    a = a_ref[0, 0]
    o_ref[...] = jnp.where(h > 0.0, h, a * h)


_RB = 400

_dense = pl.pallas_call(
    _dense_body,
    grid=(N // _RB,),
    in_specs=[
        pl.BlockSpec((NC, _RB, D), lambda i: (0, i, 0)),
        pl.BlockSpec((NC, _RB, 1), lambda i: (0, i, 0)),
        pl.BlockSpec((_RB, D), lambda i: (i, 0)),
        pl.BlockSpec((D, D), lambda i: (0, 0)),
        pl.BlockSpec((D, D), lambda i: (0, 0)),
        pl.BlockSpec((1, D), lambda i: (0, 0)),
        pl.BlockSpec((1, 1), lambda i: (0, 0)),
    ],
    out_specs=pl.BlockSpec((_RB, D), lambda i: (i, 0)),
    out_shape=jax.ShapeDtypeStruct((N, D), jnp.float32),
)


def kernel(x, edge_index, W_l1, W_r1, b1, a1, W_l2, W_r2, b2, a2):
    h0 = x[0]
    ei = edge_index[0].astype(jnp.int32)
    pad = E_PAD - E
    src = jnp.concatenate([ei[0], jnp.zeros((pad,), jnp.int32)]).reshape(NW, NBLK, 1, BLK)
    dst = jnp.concatenate([ei[1], jnp.full((pad,), N, jnp.int32)]).reshape(NW, NBLK, 1, BLK)
    srcdst = jnp.concatenate([src, dst], axis=2)

    acc1, cnt1 = _seg_sum(h0, srcdst)
    h1 = _dense(acc1, cnt1.reshape(NC, N_PAD, 1), h0, W_l1, W_r1,
                b1.reshape(1, D), a1.reshape(1, 1))
    acc2, cnt2 = _seg_sum(h1, srcdst)
    h2 = _dense(acc2, cnt2.reshape(NC, N_PAD, 1), h1, W_l2, W_r2,
                b2.reshape(1, D), a2.reshape(1, 1))
    return h2.reshape(x.shape[:-2] + (-1,))


# speedup vs baseline: 8.0185x; 1.0464x over previous
import jax
import jax.numpy as jnp
from jax import lax
from jax.experimental import pallas as pl
from jax.experimental.pallas import tpu as pltpu
from jax.experimental.pallas import tpu_sc as plsc

N = 10000
E = 320000
D = 128

NC = 2
NS = 16
NW = NC * NS
BLK = 64

EPW = ((E + NW - 1) // NW + BLK - 1) // BLK * BLK
NBLK = EPW // BLK
E_PAD = EPW * NW

ROWS_PER_TILE = 640
N_PAD = NS * ROWS_PER_TILE
ZBLK = ROWS_PER_TILE // BLK

R = 4
Q = 8


def _make_seg(with_cnt):
    def body(*refs):
        if with_cnt:
            (x_hbm, ei_hbm, acc_out, cnt_out,
             acc_sh, cnt_sh, idxb, rows, ones, zcnt, isem, gsem, ssem, osem) = refs
        else:
            (x_hbm, ei_hbm, acc_out,
             acc_sh, idxb, rows, isem, gsem, ssem) = refs
        cid = lax.axis_index("c")
        sid = lax.axis_index("s")
        wid = sid * NC + cid
        zero16 = jnp.zeros((16,), jnp.float32)

        if with_cnt:
            one16 = jnp.ones((16,), jnp.float32)
            for i in range(BLK // 16):
                ones[pl.ds(i * 16, 16)] = one16
            for i in range(ROWS_PER_TILE // 16):
                zcnt[pl.ds(i * 16, 16)] = zero16

        def _zrow(rr, carry):
            for c in range(D // 16):
                rows[0, rr, pl.ds(c * 16, 16)] = zero16
            return carry
        lax.fori_loop(0, BLK, _zrow, 0)
        base_row = sid * ROWS_PER_TILE
        for j in range(ZBLK):
            pltpu.sync_copy(rows.at[0], acc_sh.at[pl.ds(base_row + j * BLK, BLK)])
        if with_cnt:
            pltpu.sync_copy(zcnt, cnt_sh.at[pl.ds(base_row, ROWS_PER_TILE)])
        plsc.subcore_barrier()

        def _prefetch(b, q):
            pltpu.async_copy(ei_hbm.at[wid, b], idxb.at[q], isem.at[q])

        def _wait_idx(q):
            pltpu.make_async_copy(ei_hbm.at[wid, 0], idxb.at[q], isem.at[q]).wait()

        def _gather(q, r):
            pltpu.async_copy(x_hbm.at[idxb.at[q, 0]], rows.at[r], gsem.at[r])

        def _wait_rows(r):
            pltpu.make_async_copy(x_hbm.at[pl.ds(0, BLK)], rows.at[r], gsem.at[r]).wait()

        def _scatter(q, r):
            pltpu.async_copy(rows.at[r], acc_sh.at[idxb.at[q, 1]], ssem.at[r], add=True)
            if with_cnt:
                pltpu.async_copy(ones, cnt_sh.at[idxb.at[q, 1]], osem.at[r], add=True)

        def _wait_scatter(r):
            pltpu.make_async_copy(x_hbm.at[pl.ds(0, BLK)], rows.at[r], ssem.at[r]).wait()
            if with_cnt:
                pltpu.make_async_copy(x_hbm.at[0, pl.ds(0, BLK)], ones, osem.at[r]).wait()

        for b in range(6):
            _prefetch(b, b)
        _wait_idx(0)
        _gather(0, 0)
        _wait_idx(1)
        _gather(1, 1)

        def _step(b, carry):
            r = lax.rem(b, R)
            q = lax.rem(b, Q)
            _wait_rows(r)
            _scatter(q, r)

            @pl.when(b >= 2)
            def _():
                _wait_scatter(lax.rem(b + 2, R))

            @pl.when(b + 2 < NBLK)
            def _():
                q2 = lax.rem(b + 2, Q)
                _wait_idx(q2)
                _gather(q2, lax.rem(b + 2, R))

            @pl.when(b + 6 < NBLK)
            def _():
                _prefetch(b + 6, lax.rem(b + 6, Q))
            return carry
        lax.fori_loop(0, NBLK, _step, 0)
        _wait_scatter((NBLK - 2) % R)
        _wait_scatter((NBLK - 1) % R)
        plsc.subcore_barrier()

        for j in range(ZBLK):
            rr = base_row + j * BLK
            pltpu.sync_copy(acc_sh.at[pl.ds(rr, BLK)], acc_out.at[cid, pl.ds(rr, BLK)])
        if with_cnt:
            pltpu.sync_copy(cnt_sh.at[pl.ds(base_row, ROWS_PER_TILE)],
                            cnt_out.at[cid, pl.ds(base_row, ROWS_PER_TILE)])

    out_type = [jax.ShapeDtypeStruct((NC, N_PAD, D), jnp.float32)]
    scratch = [
        pltpu.VMEM_SHARED((N_PAD, D), jnp.float32),
        pltpu.VMEM((Q, 2, BLK), jnp.int32),
        pltpu.VMEM((R, BLK, D), jnp.float32),
        pltpu.SemaphoreType.DMA((Q,)),
        pltpu.SemaphoreType.DMA((R,)),
        pltpu.SemaphoreType.DMA((R,)),
    ]
    if with_cnt:
        out_type.append(jax.ShapeDtypeStruct((NC, N_PAD), jnp.float32))
        scratch = ([scratch[0],
                    pltpu.VMEM_SHARED((N_PAD,), jnp.float32)]
                   + scratch[1:3]
                   + [pltpu.VMEM((BLK,), jnp.float32),
                      pltpu.VMEM((ROWS_PER_TILE,), jnp.float32)]
                   + scratch[3:]
                   + [pltpu.SemaphoreType.DMA((R,))])

    return pl.kernel(
        body,
        out_type=out_type,
        mesh=plsc.VectorSubcoreMesh(core_axis_name="c", subcore_axis_name="s",
                                    num_cores=NC, num_subcores=NS),
        scratch_types=scratch,
    )


_seg_sum_cnt = _make_seg(True)
_seg_sum = _make_seg(False)


def _dense_body(acc_ref, cnt_ref, x_ref, wl_ref, wr_ref, b_ref, a_ref, o_ref):
    cnt = cnt_ref[0] + cnt_ref[1]
    mean = (acc_ref[0] + acc_ref[1]) / jnp.maximum(cnt, 1.0)
    h = lax.dot_general(mean, wl_ref[...], (((1,), (1,)), ((), ())),
                        preferred_element_type=jnp.float32)
    h = h + lax.dot_general(x_ref[...], wr_ref[...], (((1,), (1,)), ((), ())),
                            preferred_element_type=jnp.float32)
    h = h + b_ref[...]
    a = a_ref[0, 0]
    o_ref[...] = jnp.where(h > 0.0, h, a * h)


_RB = 400

_dense = pl.pallas_call(
    _dense_body,
    grid=(N // _RB,),
    in_specs=[
        pl.BlockSpec((NC, _RB, D), lambda i: (0, i, 0)),
        pl.BlockSpec((NC, _RB, 1), lambda i: (0, i, 0)),
        pl.BlockSpec((_RB, D), lambda i: (i, 0)),
        pl.BlockSpec((D, D), lambda i: (0, 0)),
        pl.BlockSpec((D, D), lambda i: (0, 0)),
        pl.BlockSpec((1, D), lambda i: (0, 0)),
        pl.BlockSpec((1, 1), lambda i: (0, 0)),
    ],
    out_specs=pl.BlockSpec((_RB, D), lambda i: (i, 0)),
    out_shape=jax.ShapeDtypeStruct((N, D), jnp.float32),
)


def kernel(x, edge_index, W_l1, W_r1, b1, a1, W_l2, W_r2, b2, a2):
    h0 = x[0]
    ei = edge_index[0].astype(jnp.int32)
    pad = E_PAD - E
    src = jnp.concatenate([ei[0], jnp.zeros((pad,), jnp.int32)]).reshape(NW, NBLK, 1, BLK)
    dst = jnp.concatenate([ei[1], jnp.full((pad,), N, jnp.int32)]).reshape(NW, NBLK, 1, BLK)
    srcdst = jnp.concatenate([src, dst], axis=2)

    acc1, cnt1 = _seg_sum_cnt(h0, srcdst)
    cnt3 = cnt1.reshape(NC, N_PAD, 1)
    h1 = _dense(acc1, cnt3, h0, W_l1, W_r1, b1.reshape(1, D), a1.reshape(1, 1))
    acc2, = _seg_sum(h1, srcdst)
    h2 = _dense(acc2, cnt3, h1, W_l2, W_r2, b2.reshape(1, D), a2.reshape(1, 1))
    return h2.reshape(x.shape[:-2] + (-1,))


# speedup vs baseline: 8.1893x; 1.0213x over previous
import jax
import jax.numpy as jnp
from jax import lax
from jax.experimental import pallas as pl
from jax.experimental.pallas import tpu as pltpu
from jax.experimental.pallas import tpu_sc as plsc

N = 10000
E = 320000
D = 128

NC = 2
NS = 16
NW = NC * NS
BLK = 64

EPW = ((E + NW - 1) // NW + BLK - 1) // BLK * BLK
NBLK = EPW // BLK
E_PAD = EPW * NW

ROWS_PER_TILE = 640
N_PAD = NS * ROWS_PER_TILE
ZBLK = ROWS_PER_TILE // BLK

R = 5
Q = 8


def _make_seg(with_cnt):
    def body(*refs):
        if with_cnt:
            (x_hbm, ei_hbm, acc_out, cnt_out,
             acc_sh, cnt_sh, idxb, rows, ones, zcnt, isem, gsem, ssem, osem) = refs
        else:
            (x_hbm, ei_hbm, acc_out,
             acc_sh, idxb, rows, isem, gsem, ssem) = refs
        cid = lax.axis_index("c")
        sid = lax.axis_index("s")
        wid = sid * NC + cid
        zero16 = jnp.zeros((16,), jnp.float32)

        if with_cnt:
            one16 = jnp.ones((16,), jnp.float32)
            for i in range(BLK // 16):
                ones[pl.ds(i * 16, 16)] = one16
            for i in range(ROWS_PER_TILE // 16):
                zcnt[pl.ds(i * 16, 16)] = zero16

        def _zrow(rr, carry):
            for c in range(D // 16):
                rows[0, rr, pl.ds(c * 16, 16)] = zero16
            return carry
        lax.fori_loop(0, BLK, _zrow, 0)
        base_row = sid * ROWS_PER_TILE
        for j in range(ZBLK):
            pltpu.sync_copy(rows.at[0], acc_sh.at[pl.ds(base_row + j * BLK, BLK)])
        if with_cnt:
            pltpu.sync_copy(zcnt, cnt_sh.at[pl.ds(base_row, ROWS_PER_TILE)])
        plsc.subcore_barrier()

        def _prefetch(b, q):
            pltpu.async_copy(ei_hbm.at[wid, b], idxb.at[q], isem.at[q])

        def _wait_idx(q):
            pltpu.make_async_copy(ei_hbm.at[wid, 0], idxb.at[q], isem.at[q]).wait()

        def _gather(q, r):
            pltpu.async_copy(x_hbm.at[idxb.at[q, 0]], rows.at[r], gsem.at[r])

        def _wait_rows(r):
            pltpu.make_async_copy(x_hbm.at[pl.ds(0, BLK)], rows.at[r], gsem.at[r]).wait()

        def _scatter(q, r):
            pltpu.async_copy(rows.at[r], acc_sh.at[idxb.at[q, 1]], ssem.at[r], add=True)
            if with_cnt:
                pltpu.async_copy(ones, cnt_sh.at[idxb.at[q, 1]], osem.at[r], add=True)

        def _wait_scatter(r):
            pltpu.make_async_copy(x_hbm.at[pl.ds(0, BLK)], rows.at[r], ssem.at[r]).wait()
            if with_cnt:
                pltpu.make_async_copy(x_hbm.at[0, pl.ds(0, BLK)], ones, osem.at[r]).wait()

        for b in range(6):
            _prefetch(b, b)
        for b in range(3):
            _wait_idx(b)
            _gather(b, b)

        def _step(b, carry):
            r = lax.rem(b, R)
            q = lax.rem(b, Q)
            _wait_rows(r)
            _scatter(q, r)

            @pl.when(b >= 2)
            def _():
                _wait_scatter(lax.rem(b + R - 2, R))

            @pl.when(b + 3 < NBLK)
            def _():
                q2 = lax.rem(b + 3, Q)
                _wait_idx(q2)
                _gather(q2, lax.rem(b + 3, R))

            @pl.when(b + 6 < NBLK)
            def _():
                _prefetch(b + 6, lax.rem(b + 6, Q))
            return carry
        lax.fori_loop(0, NBLK, _step, 0)
        _wait_scatter((NBLK - 2) % R)
        _wait_scatter((NBLK - 1) % R)
        plsc.subcore_barrier()

        for j in range(ZBLK):
            rr = base_row + j * BLK
            pltpu.sync_copy(acc_sh.at[pl.ds(rr, BLK)], acc_out.at[cid, pl.ds(rr, BLK)])
        if with_cnt:
            pltpu.sync_copy(cnt_sh.at[pl.ds(base_row, ROWS_PER_TILE)],
                            cnt_out.at[cid, pl.ds(base_row, ROWS_PER_TILE)])

    out_type = [jax.ShapeDtypeStruct((NC, N_PAD, D), jnp.float32)]
    scratch = [
        pltpu.VMEM_SHARED((N_PAD, D), jnp.float32),
        pltpu.VMEM((Q, 2, BLK), jnp.int32),
        pltpu.VMEM((R, BLK, D), jnp.float32),
        pltpu.SemaphoreType.DMA((Q,)),
        pltpu.SemaphoreType.DMA((R,)),
        pltpu.SemaphoreType.DMA((R,)),
    ]
    if with_cnt:
        out_type.append(jax.ShapeDtypeStruct((NC, N_PAD), jnp.float32))
        scratch = ([scratch[0],
                    pltpu.VMEM_SHARED((N_PAD,), jnp.float32)]
                   + scratch[1:3]
                   + [pltpu.VMEM((BLK,), jnp.float32),
                      pltpu.VMEM((ROWS_PER_TILE,), jnp.float32)]
                   + scratch[3:]
                   + [pltpu.SemaphoreType.DMA((R,))])

    return pl.kernel(
        body,
        out_type=out_type,
        mesh=plsc.VectorSubcoreMesh(core_axis_name="c", subcore_axis_name="s",
                                    num_cores=NC, num_subcores=NS),
        scratch_types=scratch,
    )


_seg_sum_cnt = _make_seg(True)
_seg_sum = _make_seg(False)


def _dense_body(acc_ref, cnt_ref, x_ref, wl_ref, wr_ref, b_ref, a_ref, o_ref):
    cnt = cnt_ref[0] + cnt_ref[1]
    mean = (acc_ref[0] + acc_ref[1]) / jnp.maximum(cnt, 1.0)
    h = lax.dot_general(mean, wl_ref[...], (((1,), (1,)), ((), ())),
                        preferred_element_type=jnp.float32)
    h = h + lax.dot_general(x_ref[...], wr_ref[...], (((1,), (1,)), ((), ())),
                            preferred_element_type=jnp.float32)
    h = h + b_ref[...]
    a = a_ref[0, 0]
    o_ref[...] = jnp.where(h > 0.0, h, a * h)


_RB = 400

_dense = pl.pallas_call(
    _dense_body,
    grid=(N // _RB,),
    in_specs=[
        pl.BlockSpec((NC, _RB, D), lambda i: (0, i, 0)),
        pl.BlockSpec((NC, _RB, 1), lambda i: (0, i, 0)),
        pl.BlockSpec((_RB, D), lambda i: (i, 0)),
        pl.BlockSpec((D, D), lambda i: (0, 0)),
        pl.BlockSpec((D, D), lambda i: (0, 0)),
        pl.BlockSpec((1, D), lambda i: (0, 0)),
        pl.BlockSpec((1, 1), lambda i: (0, 0)),
    ],
    out_specs=pl.BlockSpec((_RB, D), lambda i: (i, 0)),
    out_shape=jax.ShapeDtypeStruct((N, D), jnp.float32),
)


def kernel(x, edge_index, W_l1, W_r1, b1, a1, W_l2, W_r2, b2, a2):
    h0 = x[0]
    ei = edge_index[0].astype(jnp.int32)
    pad = E_PAD - E
    src = jnp.concatenate([ei[0], jnp.zeros((pad,), jnp.int32)]).reshape(NW, NBLK, 1, BLK)
    dst_pad = N + jnp.arange(pad, dtype=jnp.int32) % (N_PAD - N)
    dst = jnp.concatenate([ei[1], dst_pad]).reshape(NW, NBLK, 1, BLK)
    srcdst = jnp.concatenate([src, dst], axis=2)

    acc1, cnt1 = _seg_sum_cnt(h0, srcdst)
    cnt3 = cnt1.reshape(NC, N_PAD, 1)
    h1 = _dense(acc1, cnt3, h0, W_l1, W_r1, b1.reshape(1, D), a1.reshape(1, 1))
    acc2, = _seg_sum(h1, srcdst)
    h2 = _dense(acc2, cnt3, h1, W_l2, W_r2, b2.reshape(1, D), a2.reshape(1, 1))
    return h2.reshape(x.shape[:-2] + (-1,))


# speedup vs baseline: 12.5409x; 1.5314x over previous
import jax
import jax.numpy as jnp
from jax import lax
from jax.experimental import pallas as pl
from jax.experimental.pallas import tpu as pltpu
from jax.experimental.pallas import tpu_sc as plsc

N = 10000
E = 320000
D = 128

NC = 2
NS = 16
NW = NC * NS
BLK = 64

TB = E // BLK
B0 = TB // 2
B1 = TB - B0

ROWS_PER_TILE = 640
N_PAD = NS * ROWS_PER_TILE
ZBLK = ROWS_PER_TILE // BLK

R = 5
Q = 8


def _make_seg(with_cnt):
    def body(*refs):
        if with_cnt:
            (x_hbm, ei_hbm, acc_out, cnt_out,
             acc_sh, cnt_sh, idxb, rows, ones, zcnt, isem, gsem, ssem, osem) = refs
        else:
            (x_hbm, ei_hbm, acc_out,
             acc_sh, idxb, rows, isem, gsem, ssem) = refs
        cid = lax.axis_index("c")
        sid = lax.axis_index("s")
        bc = jnp.where(cid == 0, B0, B1)
        base = jnp.where(cid == 0, 0, B0)
        start = base + lax.div(sid * bc, 16)
        cnt = base + lax.div((sid + 1) * bc, 16) - start
        zero16 = jnp.zeros((16,), jnp.float32)

        if with_cnt:
            one16 = jnp.ones((16,), jnp.float32)
            for i in range(BLK // 16):
                ones[pl.ds(i * 16, 16)] = one16
            for i in range(ROWS_PER_TILE // 16):
                zcnt[pl.ds(i * 16, 16)] = zero16

        def _zrow(rr, carry):
            for c in range(D // 16):
                rows[0, rr, pl.ds(c * 16, 16)] = zero16
            return carry
        lax.fori_loop(0, BLK, _zrow, 0)
        base_row = sid * ROWS_PER_TILE
        for j in range(ZBLK):
            pltpu.sync_copy(rows.at[0], acc_sh.at[pl.ds(base_row + j * BLK, BLK)])
        if with_cnt:
            pltpu.sync_copy(zcnt, cnt_sh.at[pl.ds(base_row, ROWS_PER_TILE)])
        plsc.subcore_barrier()

        def _prefetch(b, q):
            pltpu.async_copy(ei_hbm.at[start + b], idxb.at[q], isem.at[q])

        def _wait_idx(q):
            pltpu.make_async_copy(ei_hbm.at[0], idxb.at[q], isem.at[q]).wait()

        def _gather(q, r):
            pltpu.async_copy(x_hbm.at[idxb.at[q, 0]], rows.at[r], gsem.at[r])

        def _wait_rows(r):
            pltpu.make_async_copy(x_hbm.at[pl.ds(0, BLK)], rows.at[r], gsem.at[r]).wait()

        def _scatter(q, r):
            pltpu.async_copy(rows.at[r], acc_sh.at[idxb.at[q, 1]], ssem.at[r], add=True)
            if with_cnt:
                pltpu.async_copy(ones, cnt_sh.at[idxb.at[q, 1]], osem.at[r], add=True)

        def _wait_scatter(r):
            pltpu.make_async_copy(x_hbm.at[pl.ds(0, BLK)], rows.at[r], ssem.at[r]).wait()
            if with_cnt:
                pltpu.make_async_copy(x_hbm.at[0, pl.ds(0, BLK)], ones, osem.at[r]).wait()

        for b in range(6):
            @pl.when(b < cnt)
            def _():
                _prefetch(b, b)
        for b in range(3):
            @pl.when(b < cnt)
            def _():
                _wait_idx(b)
                _gather(b, b)

        def _step(b, carry):
            r = lax.rem(b, R)
            q = lax.rem(b, Q)
            _wait_rows(r)
            _scatter(q, r)

            @pl.when(b >= 2)
            def _():
                _wait_scatter(lax.rem(b + R - 2, R))

            @pl.when(b + 3 < cnt)
            def _():
                q2 = lax.rem(b + 3, Q)
                _wait_idx(q2)
                _gather(q2, lax.rem(b + 3, R))

            @pl.when(b + 6 < cnt)
            def _():
                _prefetch(b + 6, lax.rem(b + 6, Q))
            return carry
        lax.fori_loop(0, cnt, _step, 0)

        @pl.when(cnt >= 2)
        def _():
            _wait_scatter(lax.rem(cnt - 2, R))

        @pl.when(cnt >= 1)
        def _():
            _wait_scatter(lax.rem(cnt - 1, R))
        plsc.subcore_barrier()

        for j in range(ZBLK):
            rr = base_row + j * BLK
            pltpu.sync_copy(acc_sh.at[pl.ds(rr, BLK)], acc_out.at[cid, pl.ds(rr, BLK)])
        if with_cnt:
            pltpu.sync_copy(cnt_sh.at[pl.ds(base_row, ROWS_PER_TILE)],
                            cnt_out.at[cid, pl.ds(base_row, ROWS_PER_TILE)])

    out_type = [jax.ShapeDtypeStruct((NC, N_PAD, D), jnp.float32)]
    scratch = [
        pltpu.VMEM_SHARED((N_PAD, D), jnp.float32),
        pltpu.VMEM((Q, 2, BLK), jnp.int32),
        pltpu.VMEM((R, BLK, D), jnp.float32),
        pltpu.SemaphoreType.DMA((Q,)),
        pltpu.SemaphoreType.DMA((R,)),
        pltpu.SemaphoreType.DMA((R,)),
    ]
    if with_cnt:
        out_type.append(jax.ShapeDtypeStruct((NC, N_PAD), jnp.float32))
        scratch = ([scratch[0],
                    pltpu.VMEM_SHARED((N_PAD,), jnp.float32)]
                   + scratch[1:3]
                   + [pltpu.VMEM((BLK,), jnp.float32),
                      pltpu.VMEM((ROWS_PER_TILE,), jnp.float32)]
                   + scratch[3:]
                   + [pltpu.SemaphoreType.DMA((R,))])

    return pl.kernel(
        body,
        out_type=out_type,
        mesh=plsc.VectorSubcoreMesh(core_axis_name="c", subcore_axis_name="s",
                                    num_cores=NC, num_subcores=NS),
        scratch_types=scratch,
    )


_seg_sum_cnt = _make_seg(True)
_seg_sum = _make_seg(False)


def _dense_body(acc_ref, cnt_ref, x_ref, wl_ref, wr_ref, b_ref, a_ref, o_ref):
    cnt = cnt_ref[0] + cnt_ref[1]
    mean = (acc_ref[0] + acc_ref[1]) / jnp.maximum(cnt, 1.0)
    h = lax.dot_general(mean, wl_ref[...], (((1,), (1,)), ((), ())),
                        preferred_element_type=jnp.float32)
    h = h + lax.dot_general(x_ref[...], wr_ref[...], (((1,), (1,)), ((), ())),
                            preferred_element_type=jnp.float32)
    h = h + b_ref[...]
    a = a_ref[0, 0]
    o_ref[...] = jnp.where(h > 0.0, h, a * h)


_RB = 400

_dense = pl.pallas_call(
    _dense_body,
    grid=(N // _RB,),
    in_specs=[
        pl.BlockSpec((NC, _RB, D), lambda i: (0, i, 0)),
        pl.BlockSpec((NC, _RB, 1), lambda i: (0, i, 0)),
        pl.BlockSpec((_RB, D), lambda i: (i, 0)),
        pl.BlockSpec((D, D), lambda i: (0, 0)),
        pl.BlockSpec((D, D), lambda i: (0, 0)),
        pl.BlockSpec((1, D), lambda i: (0, 0)),
        pl.BlockSpec((1, 1), lambda i: (0, 0)),
    ],
    out_specs=pl.BlockSpec((_RB, D), lambda i: (i, 0)),
    out_shape=jax.ShapeDtypeStruct((N, D), jnp.float32),
)


def kernel(x, edge_index, W_l1, W_r1, b1, a1, W_l2, W_r2, b2, a2):
    h0 = x[0]
    ei = edge_index[0].astype(jnp.int32)
    src = ei[0].reshape(TB, 1, BLK)
    dst = ei[1].reshape(TB, 1, BLK)
    srcdst = jnp.concatenate([src, dst], axis=1)

    acc1, cnt1 = _seg_sum_cnt(h0, srcdst)
    cnt3 = cnt1.reshape(NC, N_PAD, 1)
    h1 = _dense(acc1, cnt3, h0, W_l1, W_r1, b1.reshape(1, D), a1.reshape(1, 1))
    acc2, = _seg_sum(h1, srcdst)
    h2 = _dense(acc2, cnt3, h1, W_l2, W_r2, b2.reshape(1, D), a2.reshape(1, 1))
    return h2.reshape(x.shape[:-2] + (-1,))


# speedup vs baseline: 12.6433x; 1.0082x over previous
import jax
import jax.numpy as jnp
from jax import lax
from jax.experimental import pallas as pl
from jax.experimental.pallas import tpu as pltpu
from jax.experimental.pallas import tpu_sc as plsc

N = 10000
E = 320000
D = 128

NC = 2
NS = 16
NW = NC * NS
BLK = 64

TB = E // BLK
B0 = TB // 2
B1 = TB - B0

ROWS_PER_TILE = 640
N_PAD = NS * ROWS_PER_TILE
ZBLK = ROWS_PER_TILE // BLK

R = 5
Q = 8


def _make_seg(with_cnt):
    def body(*refs):
        if with_cnt:
            (x_hbm, ei_hbm, acc_out, cnt_out,
             acc_sh, cnt_sh, idxb, rows, ones, zcnt,
             isem, gsem, ssem, zsem, osem) = refs
        else:
            (x_hbm, ei_hbm, acc_out,
             acc_sh, idxb, rows, isem, gsem, ssem, zsem) = refs
        cid = lax.axis_index("c")
        sid = lax.axis_index("s")
        bc = jnp.where(cid == 0, B0, B1)
        base = jnp.where(cid == 0, 0, B0)
        start = base + lax.div(sid * bc, 16)
        cnt = base + lax.div((sid + 1) * bc, 16) - start
        zero16 = jnp.zeros((16,), jnp.float32)

        if with_cnt:
            one16 = jnp.ones((16,), jnp.float32)
            for i in range(BLK // 16):
                ones[pl.ds(i * 16, 16)] = one16
            for i in range(ROWS_PER_TILE // 16):
                zcnt[pl.ds(i * 16, 16)] = zero16

        def _zrow(rr, carry):
            for c in range(D // 16):
                rows[0, rr, pl.ds(c * 16, 16)] = zero16
            return carry
        lax.fori_loop(0, BLK, _zrow, 0)
        base_row = sid * ROWS_PER_TILE
        for j in range(ZBLK):
            pltpu.async_copy(rows.at[0], acc_sh.at[pl.ds(base_row + j * BLK, BLK)],
                             zsem)
        if with_cnt:
            pltpu.sync_copy(zcnt, cnt_sh.at[pl.ds(base_row, ROWS_PER_TILE)])
        for j in range(ZBLK):
            pltpu.make_async_copy(rows.at[0],
                                  acc_sh.at[pl.ds(base_row + j * BLK, BLK)],
                                  zsem).wait()
        plsc.subcore_barrier()

        def _prefetch(b, q):
            pltpu.async_copy(ei_hbm.at[start + b], idxb.at[q], isem.at[q])

        def _wait_idx(q):
            pltpu.make_async_copy(ei_hbm.at[0], idxb.at[q], isem.at[q]).wait()

        def _gather(q, r):
            pltpu.async_copy(x_hbm.at[idxb.at[q, 0]], rows.at[r], gsem.at[r])

        def _wait_rows(r):
            pltpu.make_async_copy(x_hbm.at[pl.ds(0, BLK)], rows.at[r], gsem.at[r]).wait()

        def _scatter(q, r):
            pltpu.async_copy(rows.at[r], acc_sh.at[idxb.at[q, 1]], ssem.at[r], add=True)
            if with_cnt:
                pltpu.async_copy(ones, cnt_sh.at[idxb.at[q, 1]], osem.at[r], add=True)

        def _wait_scatter(r):
            pltpu.make_async_copy(x_hbm.at[pl.ds(0, BLK)], rows.at[r], ssem.at[r]).wait()
            if with_cnt:
                pltpu.make_async_copy(x_hbm.at[0, pl.ds(0, BLK)], ones, osem.at[r]).wait()

        for b in range(6):
            @pl.when(b < cnt)
            def _():
                _prefetch(b, b)
        for b in range(3):
            @pl.when(b < cnt)
            def _():
                _wait_idx(b)
                _gather(b, b)

        def _step(b, carry):
            r = lax.rem(b, R)
            q = lax.rem(b, Q)
            _wait_rows(r)
            _scatter(q, r)

            @pl.when(b >= 2)
            def _():
                _wait_scatter(lax.rem(b + R - 2, R))

            @pl.when(b + 3 < cnt)
            def _():
                q2 = lax.rem(b + 3, Q)
                _wait_idx(q2)
                _gather(q2, lax.rem(b + 3, R))

            @pl.when(b + 6 < cnt)
            def _():
                _prefetch(b + 6, lax.rem(b + 6, Q))
            return carry
        lax.fori_loop(0, cnt, _step, 0)

        @pl.when(cnt >= 2)
        def _():
            _wait_scatter(lax.rem(cnt - 2, R))

        @pl.when(cnt >= 1)
        def _():
            _wait_scatter(lax.rem(cnt - 1, R))
        plsc.subcore_barrier()

        for j in range(ZBLK):
            rr = base_row + j * BLK
            pltpu.async_copy(acc_sh.at[pl.ds(rr, BLK)],
                             acc_out.at[cid, pl.ds(rr, BLK)], zsem)
        if with_cnt:
            pltpu.sync_copy(cnt_sh.at[pl.ds(base_row, ROWS_PER_TILE)],
                            cnt_out.at[cid, pl.ds(base_row, ROWS_PER_TILE)])
        for j in range(ZBLK):
            rr = base_row + j * BLK
            pltpu.make_async_copy(acc_sh.at[pl.ds(rr, BLK)],
                                  acc_out.at[cid, pl.ds(rr, BLK)], zsem).wait()

    out_type = [jax.ShapeDtypeStruct((NC, N_PAD, D), jnp.float32)]
    scratch = [
        pltpu.VMEM_SHARED((N_PAD, D), jnp.float32),
        pltpu.VMEM((Q, 2, BLK), jnp.int32),
        pltpu.VMEM((R, BLK, D), jnp.float32),
        pltpu.SemaphoreType.DMA((Q,)),
        pltpu.SemaphoreType.DMA((R,)),
        pltpu.SemaphoreType.DMA((R,)),
        pltpu.SemaphoreType.DMA,
    ]
    if with_cnt:
        out_type.append(jax.ShapeDtypeStruct((NC, N_PAD), jnp.float32))
        scratch = ([scratch[0],
                    pltpu.VMEM_SHARED((N_PAD,), jnp.float32)]
                   + scratch[1:3]
                   + [pltpu.VMEM((BLK,), jnp.float32),
                      pltpu.VMEM((ROWS_PER_TILE,), jnp.float32)]
                   + scratch[3:]
                   + [pltpu.SemaphoreType.DMA((R,))])

    return pl.kernel(
        body,
        out_type=out_type,
        mesh=plsc.VectorSubcoreMesh(core_axis_name="c", subcore_axis_name="s",
                                    num_cores=NC, num_subcores=NS),
        scratch_types=scratch,
    )


_seg_sum_cnt = _make_seg(True)
_seg_sum = _make_seg(False)


_RB = 400


def _self_body(x_ref, wr_ref, b_ref, o_ref):
    o_ref[...] = lax.dot_general(
        x_ref[...], wr_ref[...], (((1,), (1,)), ((), ())),
        preferred_element_type=jnp.float32) + b_ref[...]


_self_dense = pl.pallas_call(
    _self_body,
    grid=(N // _RB,),
    in_specs=[
        pl.BlockSpec((_RB, D), lambda i: (i, 0)),
        pl.BlockSpec((D, D), lambda i: (0, 0)),
        pl.BlockSpec((1, D), lambda i: (0, 0)),
    ],
    out_specs=pl.BlockSpec((_RB, D), lambda i: (i, 0)),
    out_shape=jax.ShapeDtypeStruct((N, D), jnp.float32),
)


def _combine_body(acc_ref, cnt_ref, s_ref, wl_ref, a_ref, o_ref):
    cnt = cnt_ref[0] + cnt_ref[1]
    mean = (acc_ref[0] + acc_ref[1]) / jnp.maximum(cnt, 1.0)
    h = lax.dot_general(mean, wl_ref[...], (((1,), (1,)), ((), ())),
                        preferred_element_type=jnp.float32) + s_ref[...]
    a = a_ref[0, 0]
    o_ref[...] = jnp.where(h > 0.0, h, a * h)


_combine = pl.pallas_call(
    _combine_body,
    grid=(N // _RB,),
    in_specs=[
        pl.BlockSpec((NC, _RB, D), lambda i: (0, i, 0)),
        pl.BlockSpec((NC, _RB, 1), lambda i: (0, i, 0)),
        pl.BlockSpec((_RB, D), lambda i: (i, 0)),
        pl.BlockSpec((D, D), lambda i: (0, 0)),
        pl.BlockSpec((1, 1), lambda i: (0, 0)),
    ],
    out_specs=pl.BlockSpec((_RB, D), lambda i: (i, 0)),
    out_shape=jax.ShapeDtypeStruct((N, D), jnp.float32),
)


def kernel(x, edge_index, W_l1, W_r1, b1, a1, W_l2, W_r2, b2, a2):
    h0 = x[0]
    ei = edge_index[0].astype(jnp.int32)
    src = ei[0].reshape(TB, 1, BLK)
    dst = ei[1].reshape(TB, 1, BLK)
    srcdst = jnp.concatenate([src, dst], axis=1)

    acc1, cnt1 = _seg_sum_cnt(h0, srcdst)
    self1 = _self_dense(h0, W_r1, b1.reshape(1, D))
    cnt3 = cnt1.reshape(NC, N_PAD, 1)
    h1 = _combine(acc1, cnt3, self1, W_l1, a1.reshape(1, 1))
    acc2, = _seg_sum(h1, srcdst)
    self2 = _self_dense(h1, W_r2, b2.reshape(1, D))
    h2 = _combine(acc2, cnt3, self2, W_l2, a2.reshape(1, 1))
    return h2.reshape(x.shape[:-2] + (-1,))


# speedup vs baseline: 14.4054x; 1.1394x over previous
import jax
import jax.numpy as jnp
from jax import lax
from jax.experimental import pallas as pl
from jax.experimental.pallas import tpu as pltpu
from jax.experimental.pallas import tpu_sc as plsc

N = 10000
E = 320000
D = 128

NC = 2
NS = 16
NW = NC * NS
BLK = 64

TB = E // BLK
B0 = TB // 2
B1 = TB - B0

ROWS_PER_TILE = 640
N_PAD = NS * ROWS_PER_TILE
ZBLK = ROWS_PER_TILE // BLK

R = 5
Q = 8


def _make_seg(with_cnt):
    def body(*refs):
        if with_cnt:
            (x_hbm, ei_hbm, acc_out, cnt_out,
             acc_sh, cnt_sh, idxb, rows, ones, zcnt,
             isem, gsem, ssem, zsem, osem) = refs
        else:
            (x_hbm, ei_hbm, acc_out,
             acc_sh, idxb, rows, isem, gsem, ssem, zsem) = refs
        cid = lax.axis_index("c")
        sid = lax.axis_index("s")
        bc = jnp.where(cid == 0, B0, B1)
        base = jnp.where(cid == 0, 0, B0)
        start = base + lax.div(sid * bc, 16)
        cnt = base + lax.div((sid + 1) * bc, 16) - start
        zero16 = jnp.zeros((16,), jnp.float32)

        if with_cnt:
            one16 = jnp.ones((16,), jnp.float32)
            for i in range(BLK // 16):
                ones[pl.ds(i * 16, 16)] = one16
            for i in range(ROWS_PER_TILE // 16):
                zcnt[pl.ds(i * 16, 16)] = zero16

        def _zrow(rr, carry):
            for c in range(D // 16):
                rows[0, rr, pl.ds(c * 16, 16)] = zero16
            return carry
        lax.fori_loop(0, BLK, _zrow, 0)
        base_row = sid * ROWS_PER_TILE
        for j in range(ZBLK):
            pltpu.async_copy(rows.at[0], acc_sh.at[pl.ds(base_row + j * BLK, BLK)],
                             zsem)
        if with_cnt:
            pltpu.sync_copy(zcnt, cnt_sh.at[pl.ds(base_row, ROWS_PER_TILE)])
        for j in range(ZBLK):
            pltpu.make_async_copy(rows.at[0],
                                  acc_sh.at[pl.ds(base_row + j * BLK, BLK)],
                                  zsem).wait()
        plsc.subcore_barrier()

        def _prefetch(b, q):
            off = (start + b) * BLK
            pltpu.async_copy(ei_hbm.at[0, pl.ds(off, BLK)], idxb.at[q, 0], isem.at[q])
            pltpu.async_copy(ei_hbm.at[1, pl.ds(off, BLK)], idxb.at[q, 1], isem.at[q])

        def _wait_idx(q):
            pltpu.make_async_copy(ei_hbm.at[0, pl.ds(0, BLK)], idxb.at[q, 0], isem.at[q]).wait()
            pltpu.make_async_copy(ei_hbm.at[1, pl.ds(0, BLK)], idxb.at[q, 1], isem.at[q]).wait()

        def _gather(q, r):
            pltpu.async_copy(x_hbm.at[idxb.at[q, 0]], rows.at[r], gsem.at[r])

        def _wait_rows(r):
            pltpu.make_async_copy(x_hbm.at[pl.ds(0, BLK)], rows.at[r], gsem.at[r]).wait()

        def _scatter(q, r):
            pltpu.async_copy(rows.at[r], acc_sh.at[idxb.at[q, 1]], ssem.at[r], add=True)
            if with_cnt:
                pltpu.async_copy(ones, cnt_sh.at[idxb.at[q, 1]], osem.at[r], add=True)

        def _wait_scatter(r):
            pltpu.make_async_copy(x_hbm.at[pl.ds(0, BLK)], rows.at[r], ssem.at[r]).wait()
            if with_cnt:
                pltpu.make_async_copy(x_hbm.at[0, pl.ds(0, BLK)], ones, osem.at[r]).wait()

        for b in range(6):
            @pl.when(b < cnt)
            def _():
                _prefetch(b, b)
        for b in range(3):
            @pl.when(b < cnt)
            def _():
                _wait_idx(b)
                _gather(b, b)

        def _step(b, carry):
            r = lax.rem(b, R)
            q = lax.rem(b, Q)
            _wait_rows(r)
            _scatter(q, r)

            @pl.when(b >= 2)
            def _():
                _wait_scatter(lax.rem(b + R - 2, R))

            @pl.when(b + 3 < cnt)
            def _():
                q2 = lax.rem(b + 3, Q)
                _wait_idx(q2)
                _gather(q2, lax.rem(b + 3, R))

            @pl.when(b + 6 < cnt)
            def _():
                _prefetch(b + 6, lax.rem(b + 6, Q))
            return carry
        lax.fori_loop(0, cnt, _step, 0)

        @pl.when(cnt >= 2)
        def _():
            _wait_scatter(lax.rem(cnt - 2, R))

        @pl.when(cnt >= 1)
        def _():
            _wait_scatter(lax.rem(cnt - 1, R))
        plsc.subcore_barrier()

        for j in range(ZBLK):
            rr = base_row + j * BLK
            pltpu.async_copy(acc_sh.at[pl.ds(rr, BLK)],
                             acc_out.at[cid, pl.ds(rr, BLK)], zsem)
        if with_cnt:
            pltpu.sync_copy(cnt_sh.at[pl.ds(base_row, ROWS_PER_TILE)],
                            cnt_out.at[cid, pl.ds(base_row, ROWS_PER_TILE)])
        for j in range(ZBLK):
            rr = base_row + j * BLK
            pltpu.make_async_copy(acc_sh.at[pl.ds(rr, BLK)],
                                  acc_out.at[cid, pl.ds(rr, BLK)], zsem).wait()

    out_type = [jax.ShapeDtypeStruct((NC, N_PAD, D), jnp.float32)]
    scratch = [
        pltpu.VMEM_SHARED((N_PAD, D), jnp.float32),
        pltpu.VMEM((Q, 2, BLK), jnp.int32),
        pltpu.VMEM((R, BLK, D), jnp.float32),
        pltpu.SemaphoreType.DMA((Q,)),
        pltpu.SemaphoreType.DMA((R,)),
        pltpu.SemaphoreType.DMA((R,)),
        pltpu.SemaphoreType.DMA,
    ]
    if with_cnt:
        out_type.append(jax.ShapeDtypeStruct((NC, N_PAD), jnp.float32))
        scratch = ([scratch[0],
                    pltpu.VMEM_SHARED((N_PAD,), jnp.float32)]
                   + scratch[1:3]
                   + [pltpu.VMEM((BLK,), jnp.float32),
                      pltpu.VMEM((ROWS_PER_TILE,), jnp.float32)]
                   + scratch[3:]
                   + [pltpu.SemaphoreType.DMA((R,))])

    return pl.kernel(
        body,
        out_type=out_type,
        mesh=plsc.VectorSubcoreMesh(core_axis_name="c", subcore_axis_name="s",
                                    num_cores=NC, num_subcores=NS),
        scratch_types=scratch,
    )


_seg_sum_cnt = _make_seg(True)
_seg_sum = _make_seg(False)


_RB = 400


def _self_body(x_ref, wr_ref, b_ref, o_ref):
    o_ref[...] = lax.dot_general(
        x_ref[...], wr_ref[...], (((1,), (1,)), ((), ())),
        preferred_element_type=jnp.float32) + b_ref[...]


_self_dense = pl.pallas_call(
    _self_body,
    grid=(N // _RB,),
    in_specs=[
        pl.BlockSpec((_RB, D), lambda i: (i, 0)),
        pl.BlockSpec((D, D), lambda i: (0, 0)),
        pl.BlockSpec((1, D), lambda i: (0, 0)),
    ],
    out_specs=pl.BlockSpec((_RB, D), lambda i: (i, 0)),
    out_shape=jax.ShapeDtypeStruct((N, D), jnp.float32),
)


def _combine_body(acc_ref, cnt_ref, s_ref, wl_ref, a_ref, o_ref):
    cnt = cnt_ref[0] + cnt_ref[1]
    mean = (acc_ref[0] + acc_ref[1]) / jnp.maximum(cnt, 1.0)
    h = lax.dot_general(mean, wl_ref[...], (((1,), (1,)), ((), ())),
                        preferred_element_type=jnp.float32) + s_ref[...]
    a = a_ref[0, 0]
    o_ref[...] = jnp.where(h > 0.0, h, a * h)


_combine = pl.pallas_call(
    _combine_body,
    grid=(N // _RB,),
    in_specs=[
        pl.BlockSpec((NC, _RB, D), lambda i: (0, i, 0)),
        pl.BlockSpec((NC, _RB, 1), lambda i: (0, i, 0)),
        pl.BlockSpec((_RB, D), lambda i: (i, 0)),
        pl.BlockSpec((D, D), lambda i: (0, 0)),
        pl.BlockSpec((1, 1), lambda i: (0, 0)),
    ],
    out_specs=pl.BlockSpec((_RB, D), lambda i: (i, 0)),
    out_shape=jax.ShapeDtypeStruct((N, D), jnp.float32),
)


def kernel(x, edge_index, W_l1, W_r1, b1, a1, W_l2, W_r2, b2, a2):
    h0 = x[0]
    ei = edge_index[0].astype(jnp.int32)

    acc1, cnt1 = _seg_sum_cnt(h0, ei)
    self1 = _self_dense(h0, W_r1, b1.reshape(1, D))
    cnt3 = cnt1.reshape(NC, N_PAD, 1)
    h1 = _combine(acc1, cnt3, self1, W_l1, a1.reshape(1, 1))
    acc2, = _seg_sum(h1, ei)
    self2 = _self_dense(h1, W_r2, b2.reshape(1, D))
    h2 = _combine(acc2, cnt3, self2, W_l2, a2.reshape(1, 1))
    return h2.reshape(x.shape[:-2] + (-1,))
